# baseline (speedup 1.0000x reference)
import functools
import jax
import jax.numpy as jnp
from jax import lax
from jax.experimental import pallas as pl
from jax.experimental.pallas import tpu as pltpu
from jax.experimental.pallas import tpu_sc as plsc

N = 20000
E = 100000
IN_CH = 32
OUT_CH = 32
HID = 6
RADIUS = 0.2

NC = 2
NS = 16
NW = NC * NS
B = 96
CHUNKS = 36
EPW = B * CHUNKS
EPAD = NW * EPW
NYPAD = 20480
YW = (HID + 1) * OUT_CH
YW2 = YW + 16
MW = 48
NACC = 20480
RPT = NACC // NS


def _mm_body(x_ref, w_ref, c_ref, o_ref):
    y = jnp.dot(x_ref[...], w_ref[...],
                preferred_element_type=jnp.float32,
                precision=jax.lax.Precision.HIGHEST)
    o_ref[...] = jnp.concatenate([y, c_ref[...]], axis=1)


def _fin_body(p_ref, o_ref):
    s = p_ref[0] + p_ref[1]
    cnt = jnp.maximum(s[:, OUT_CH:OUT_CH + 1], 1.0)
    o_ref[...] = s[:, :OUT_CH] / cnt


def _sc_body(y_hbm, coords_hbm, src_hbm, dst_hbm, const_hbm, part_hbm,
             cd0_v, rows0_v, msg0_v, cd1_v, rows1_v, msg1_v,
             sidx_v, didx_v, const_v, outw,
             sem_a0, sem_c0, sem_a1, sem_c1):
    cid = lax.axis_index("c")
    sid = lax.axis_index("s")
    wid = cid * NS + sid
    bufs = ((cd0_v, rows0_v, msg0_v, sem_a0, sem_c0),
            (cd1_v, rows1_v, msg1_v, sem_a1, sem_c1))

    zero = jnp.zeros((16,), jnp.float32)

    def zrow(i, _):
        for m in (msg0_v, msg1_v):
            m[i, pl.ds(0, 16)] = zero
            m[i, pl.ds(16, 16)] = zero
            m[i, pl.ds(32, 16)] = zero
        return 0
    lax.fori_loop(0, B, zrow, 0)

    base = sid * RPT

    def zchunk(t, _):
        pltpu.sync_copy(msg0_v.at[pl.ds(0, 80)],
                        outw.at[pl.ds(base + t * 80, 80)])
        return 0
    lax.fori_loop(0, RPT // 80, zchunk, 0)

    pltpu.sync_copy(const_hbm, const_v)
    pltpu.sync_copy(src_hbm.at[wid], sidx_v)
    pltpu.sync_copy(dst_hbm.at[wid], didx_v)
    w1a = const_v[0]
    w1b = const_v[1]
    b1v = const_v[2]
    oneh = const_v[3]
    oneh0 = const_v[4]

    plsc.subcore_barrier()

    def issue(j, buf):
        cd_v, rows_v, _, sem_a, sem_c = buf
        return (pltpu.async_copy(coords_hbm.at[didx_v.at[j]], cd_v, sem_a),
                pltpu.async_copy(y_hbm.at[sidx_v.at[j]], rows_v, sem_c))

    def compute(j, buf):
        cd_v, rows_v, msg_v, _, _ = buf

        def _edge_loop(e, carry):
            _edge_body(e)
            return carry

        def _edge_body(e):
            dv = cd_v[e] - rows_v[e, pl.ds(YW, 16)]
            av = jnp.abs(dv)
            a0 = jnp.broadcast_to(av[0], (16,))
            a1 = jnp.broadcast_to(av[1], (16,))
            sv = jnp.where(jnp.maximum(a0, a1) < RADIUS,
                           1.0, 0.0).astype(jnp.float32)
            hv = jnp.maximum(w1a * dv[0] + w1b * dv[1] + b1v, 0.0)
            cvec = (hv + oneh) * sv
            c6 = cvec[HID]
            acc0 = rows_v[e, pl.ds(HID * 32, 16)] * c6
            acc1 = rows_v[e, pl.ds(HID * 32 + 16, 16)] * c6
            for k in range(HID):
                ck = cvec[k]
                acc0 = acc0 + rows_v[e, pl.ds(k * 32, 16)] * ck
                acc1 = acc1 + rows_v[e, pl.ds(k * 32 + 16, 16)] * ck
            msg_v[e, pl.ds(0, 16)] = acc0
            msg_v[e, pl.ds(16, 16)] = acc1
            msg_v[e, pl.ds(OUT_CH, 16)] = oneh0 * sv

        lax.fori_loop(0, B, _edge_loop, 0, unroll=2)

    def scatter(j, buf):
        msg_v = buf[2]
        pltpu.sync_copy(msg_v, outw.at[didx_v.at[j]], add=True)

    def step(t, _):
        j0 = 2 * t
        d0 = issue(j0, bufs[0])
        d1 = issue(j0 + 1, bufs[1])
        for d in d0 + d1:
            d.wait()
        compute(j0, bufs[0])
        scatter(j0, bufs[0])
        compute(j0 + 1, bufs[1])
        scatter(j0 + 1, bufs[1])
        return 0
    lax.fori_loop(0, CHUNKS // 2, step, 0)

    plsc.subcore_barrier()

    pltpu.sync_copy(outw.at[pl.ds(base, RPT)],
                    part_hbm.at[cid, pl.ds(base, RPT)])


@jax.jit
def _run(x, coords, edge_index, W1, b1, W2, b2):
    f32 = jnp.float32
    Wbig = jnp.concatenate(
        [W2.reshape(HID, IN_CH, OUT_CH).transpose(1, 0, 2).reshape(IN_CH, HID * OUT_CH),
         b2.reshape(IN_CH, OUT_CH)], axis=1)

    coordsp = jnp.zeros((NYPAD, 16), f32)
    coordsp = coordsp.at[:N, :2].set(coords)
    coordsp = coordsp.at[N:, 0].set(1e9)

    xp = jnp.pad(x, ((0, NYPAD - N), (0, 0)))
    Y = pl.pallas_call(
        _mm_body,
        grid=(NYPAD // 1024,),
        in_specs=[pl.BlockSpec((1024, IN_CH), lambda i: (i, 0)),
                  pl.BlockSpec((IN_CH, YW), lambda i: (0, 0)),
                  pl.BlockSpec((1024, 16), lambda i: (i, 0))],
        out_specs=pl.BlockSpec((1024, YW2), lambda i: (i, 0)),
        out_shape=jax.ShapeDtypeStruct((NYPAD, YW2), f32),
    )(xp, Wbig, coordsp)

    src = edge_index[0].astype(jnp.int32)
    dst = edge_index[1].astype(jnp.int32)
    pad = EPAD - E
    srcp = jnp.concatenate([src, jnp.full((pad,), N, jnp.int32)]).reshape(NW, CHUNKS, B)
    dstp = jnp.concatenate([dst, jnp.zeros((pad,), jnp.int32)]).reshape(NW, CHUNKS, B)

    consts = jnp.zeros((8, 16), f32)
    consts = consts.at[0, :HID].set(W1[0])
    consts = consts.at[1, :HID].set(W1[1])
    consts = consts.at[2, :HID].set(b1)
    consts = consts.at[3, HID].set(1.0)
    consts = consts.at[4, 0].set(1.0)

    mesh = plsc.VectorSubcoreMesh(core_axis_name="c", subcore_axis_name="s")
    partials = pl.kernel(
        _sc_body,
        out_type=jax.ShapeDtypeStruct((NC, NACC, MW), f32),
        mesh=mesh,
        compiler_params=pltpu.CompilerParams(use_tc_tiling_on_sc=False),
        scratch_types=(
            [pltpu.VMEM((B, 16), f32),
             pltpu.VMEM((B, YW2), f32),
             pltpu.VMEM((B, MW), f32),
             pltpu.VMEM((B, 16), f32),
             pltpu.VMEM((B, YW2), f32),
             pltpu.VMEM((B, MW), f32),
             pltpu.VMEM((CHUNKS, B), jnp.int32),
             pltpu.VMEM((CHUNKS, B), jnp.int32),
             pltpu.VMEM((8, 16), f32),
             pltpu.VMEM_SHARED((NACC, MW), f32)]
            + [pltpu.SemaphoreType.DMA] * 4),
    )(Y, coordsp, srcp, dstp, consts)

    out = pl.pallas_call(
        _fin_body,
        grid=(N // 1000,),
        in_specs=[pl.BlockSpec((NC, 1000, MW), lambda i: (0, i, 0))],
        out_specs=pl.BlockSpec((1000, OUT_CH), lambda i: (i, 0)),
        out_shape=jax.ShapeDtypeStruct((N, OUT_CH), f32),
    )(partials)
    return out


def kernel(x, coords, edge_index, W1, b1, W2, b2):
    return _run(x, coords, edge_index, W1, b1, W2, b2)


# speedup vs baseline: 1.5854x; 1.5854x over previous
import functools
import jax
import jax.numpy as jnp
from jax import lax
from jax.experimental import pallas as pl
from jax.experimental.pallas import tpu as pltpu
from jax.experimental.pallas import tpu_sc as plsc

N = 20000
E = 100000
IN_CH = 32
OUT_CH = 32
HID = 6
RADIUS = 0.2

NC = 2
NS = 16
NW = NC * NS
B = 128
CHUNKS = 25
EPW = B * CHUNKS
EPAD = NW * EPW
NYPAD = 20480
YW = (HID + 1) * OUT_CH
YW2 = YW + 16
MW = 48
NACC = 20480
RPT = NACC // NS


def _mm_body(x_ref, w_ref, c_ref, o_ref):
    y = jnp.dot(x_ref[...], w_ref[...],
                preferred_element_type=jnp.float32,
                precision=jax.lax.Precision.HIGHEST)
    o_ref[...] = jnp.concatenate([y, c_ref[...]], axis=1)


def _fin_body(p_ref, o_ref):
    s = p_ref[0] + p_ref[1]
    cnt = jnp.maximum(s[:, OUT_CH:OUT_CH + 1], 1.0)
    o_ref[...] = s[:, :OUT_CH] / cnt


def _sc_body(y_hbm, coords_hbm, src_hbm, dst_hbm, const_hbm, part_hbm,
             cd_v, rows_v, msg_v, sidx_v, didx_v, const_v, outw,
             sem_a, sem_c):
    cid = lax.axis_index("c")
    sid = lax.axis_index("s")
    wid = cid * NS + sid
    buf = (cd_v, rows_v, msg_v, sem_a, sem_c)

    zero = jnp.zeros((16,), jnp.float32)

    def zrow(i, _):
        msg_v[i, pl.ds(0, 16)] = zero
        msg_v[i, pl.ds(16, 16)] = zero
        msg_v[i, pl.ds(32, 16)] = zero
        return 0
    lax.fori_loop(0, B, zrow, 0)

    base = sid * RPT

    def zchunk(t, _):
        pltpu.sync_copy(msg_v, outw.at[pl.ds(base + t * B, B)])
        return 0
    lax.fori_loop(0, RPT // B, zchunk, 0)

    pltpu.sync_copy(const_hbm, const_v)
    pltpu.sync_copy(src_hbm.at[wid], sidx_v)
    pltpu.sync_copy(dst_hbm.at[wid], didx_v)
    w1a = const_v[0]
    w1b = const_v[1]
    b1v = const_v[2]
    oneh = const_v[3]
    oneh0 = const_v[4]

    plsc.subcore_barrier()

    def issue(j, b):
        cd_v, rows_v, _, sem_a, sem_c = b
        return (pltpu.async_copy(coords_hbm.at[didx_v.at[j]], cd_v, sem_a),
                pltpu.async_copy(y_hbm.at[sidx_v.at[j]], rows_v, sem_c))

    def compute(j, b):
        cd_v, rows_v, msg_v, _, _ = b

        def _edge_loop(e, carry):
            _edge_body(e)
            return carry

        def _edge_body(e):
            dv = cd_v[e] - rows_v[e, pl.ds(YW, 16)]
            av = jnp.abs(dv)
            a0 = jnp.broadcast_to(av[0], (16,))
            a1 = jnp.broadcast_to(av[1], (16,))
            sv = jnp.where(jnp.maximum(a0, a1) < RADIUS,
                           1.0, 0.0).astype(jnp.float32)
            hv = jnp.maximum(w1a * dv[0] + w1b * dv[1] + b1v, 0.0)
            cvec = (hv + oneh) * sv
            c6 = cvec[HID]
            acc0 = rows_v[e, pl.ds(HID * 32, 16)] * c6
            acc1 = rows_v[e, pl.ds(HID * 32 + 16, 16)] * c6
            for k in range(HID):
                ck = cvec[k]
                acc0 = acc0 + rows_v[e, pl.ds(k * 32, 16)] * ck
                acc1 = acc1 + rows_v[e, pl.ds(k * 32 + 16, 16)] * ck
            msg_v[e, pl.ds(0, 16)] = acc0
            msg_v[e, pl.ds(16, 16)] = acc1
            msg_v[e, pl.ds(OUT_CH, 16)] = oneh0 * sv

        lax.fori_loop(0, B, _edge_loop, 0, unroll=2)

    def scatter(j, b):
        msg_v = b[2]
        pltpu.sync_copy(msg_v, outw.at[didx_v.at[j]], add=True)

    def step(j, _):
        for d in issue(j, buf):
            d.wait()
        compute(j, buf)
        scatter(j, buf)
        return 0
    lax.fori_loop(0, CHUNKS, step, 0)

    plsc.subcore_barrier()

    pltpu.sync_copy(outw.at[pl.ds(base, RPT)],
                    part_hbm.at[cid, pl.ds(base, RPT)])


@jax.jit
def _run(x, coords, edge_index, W1, b1, W2, b2):
    f32 = jnp.float32
    Wbig = jnp.concatenate(
        [W2.reshape(HID, IN_CH, OUT_CH).transpose(1, 0, 2).reshape(IN_CH, HID * OUT_CH),
         b2.reshape(IN_CH, OUT_CH)], axis=1)

    coordsp = jnp.zeros((NYPAD, 16), f32)
    coordsp = coordsp.at[:N, :2].set(coords)
    coordsp = coordsp.at[N:, 0].set(1e9)

    xp = jnp.pad(x, ((0, NYPAD - N), (0, 0)))
    Y = pl.pallas_call(
        _mm_body,
        grid=(NYPAD // 1024,),
        in_specs=[pl.BlockSpec((1024, IN_CH), lambda i: (i, 0)),
                  pl.BlockSpec((IN_CH, YW), lambda i: (0, 0)),
                  pl.BlockSpec((1024, 16), lambda i: (i, 0))],
        out_specs=pl.BlockSpec((1024, YW2), lambda i: (i, 0)),
        out_shape=jax.ShapeDtypeStruct((NYPAD, YW2), f32),
    )(xp, Wbig, coordsp)

    src = edge_index[0].astype(jnp.int32)
    dst = edge_index[1].astype(jnp.int32)
    pad = EPAD - E
    srcp = jnp.concatenate([src, jnp.full((pad,), N, jnp.int32)]).reshape(NW, CHUNKS, B)
    dstp = jnp.concatenate([dst, jnp.zeros((pad,), jnp.int32)]).reshape(NW, CHUNKS, B)

    consts = jnp.zeros((8, 16), f32)
    consts = consts.at[0, :HID].set(W1[0])
    consts = consts.at[1, :HID].set(W1[1])
    consts = consts.at[2, :HID].set(b1)
    consts = consts.at[3, HID].set(1.0)
    consts = consts.at[4, 0].set(1.0)

    mesh = plsc.VectorSubcoreMesh(core_axis_name="c", subcore_axis_name="s")
    partials = pl.kernel(
        _sc_body,
        out_type=jax.ShapeDtypeStruct((NC, NACC, MW), f32),
        mesh=mesh,
        compiler_params=pltpu.CompilerParams(use_tc_tiling_on_sc=False),
        scratch_types=(
            [pltpu.VMEM((B, 16), f32),
             pltpu.VMEM((B, YW2), f32),
             pltpu.VMEM((B, MW), f32),
             pltpu.VMEM((CHUNKS, B), jnp.int32),
             pltpu.VMEM((CHUNKS, B), jnp.int32),
             pltpu.VMEM((8, 16), f32),
             pltpu.VMEM_SHARED((NACC, MW), f32)]
            + [pltpu.SemaphoreType.DMA] * 2),
    )(Y, coordsp, srcp, dstp, consts)

    out = pl.pallas_call(
        _fin_body,
        grid=(N // 1000,),
        in_specs=[pl.BlockSpec((NC, 1000, MW), lambda i: (0, i, 0))],
        out_specs=pl.BlockSpec((1000, OUT_CH), lambda i: (i, 0)),
        out_shape=jax.ShapeDtypeStruct((N, OUT_CH), f32),
    )(partials)
    return out


def kernel(x, coords, edge_index, W1, b1, W2, b2):
    return _run(x, coords, edge_index, W1, b1, W2, b2)
